# baseline (speedup 1.0000x reference)
import functools

import jax
LANES = 16
import jax.numpy as jnp
from jax import lax
from jax.experimental import pallas as pl
from jax.experimental.pallas import tpu as pltpu
from jax.experimental.pallas import tpu_sc as plsc

MAXLEN = 200
EMBED = 64
BATCH = 4096
VOCAB = 100000

_info = plsc.get_sparse_core_info()
NC = _info.num_cores
NS = _info.num_subcores
NW = NC * NS

BCH = 512
NBC = BATCH // BCH
ITEMS = MAXLEN * NBC
IPW = ITEMS // NW
NBUF = 3
ROUNDS = IPW // NBUF
REM = IPW - ROUNDS * NBUF


def _gather_body(xt_hbm, tok_hbm, mid_hbm,
                 i0, i1, i2, r0, r1, r2,
                 g0, g1, g2, w0, w1, w2, s0, s1, s2):
    idx = (i0, i1, i2)
    rows = (r0, r1, r2)
    gsem = (g0, g1, g2)
    wsem = (w0, w1, w2)
    isem = (s0, s1, s2)

    wid = lax.axis_index("s") * NC + lax.axis_index("c")
    ibase = wid * IPW

    def fire_idx(p, it):
        m = it // NBC
        c = it % NBC
        pltpu.async_copy(xt_hbm.at[m, pl.ds(c * BCH, BCH)], idx[p], isem[p])

    def fire_gather(p, it):
        del it
        pltpu.make_async_copy(xt_hbm.at[0, pl.ds(0, BCH)], idx[p],
                              isem[p]).wait()
        pltpu.async_copy(tok_hbm.at[idx[p]], rows[p], gsem[p])

    def fire_wb(p, it):
        pltpu.make_async_copy(tok_hbm.at[pl.ds(0, BCH)], rows[p],
                              gsem[p]).wait()
        m = it // NBC
        c = it % NBC
        base = m * BATCH + c * BCH
        pltpu.async_copy(rows[p], mid_hbm.at[pl.ds(base, BCH)], wsem[p])

    def drain_wb(p):
        pltpu.make_async_copy(rows[p], mid_hbm.at[pl.ds(0, BCH)],
                              wsem[p]).wait()

    fire_idx(0, ibase)
    fire_idx(1, ibase + 1)
    fire_gather(0, ibase)

    def round_body(rnd, _):
        for p in range(NBUF):
            jj = rnd * NBUF + p
            it = ibase + jj
            q = (p + 1) % NBUF
            s = (p + 2) % NBUF

            def _prep(s=s, it=it, jj=jj):
                pl.when(jj >= 1)(lambda: drain_wb(s))
                fire_idx(s, it + 2)

            def _next(q=q, it=it):
                fire_gather(q, it + 1)

            pl.when(jj + 2 < IPW)(_prep)
            pl.when(jj + 1 < IPW)(_next)
            fire_wb(p, it)
        return ()

    lax.fori_loop(0, ROUNDS, round_body, ())
    for j in range(REM):
        jj = ROUNDS * NBUF + j
        p = jj % NBUF
        q = (p + 1) % NBUF
        if j + 1 < REM:
            pl.when(True)(lambda q=q: fire_gather(q, ibase + jj + 1))
        fire_wb(p, ibase + jj)

    for p in range(NBUF):
        drain_wb(p)


def _addpos_body(mid_ref, pos_ref, eye_ref, out_ref):
    m = pl.program_id(0) // NBC
    g = mid_ref[...].reshape(BCH, EMBED)
    p = pos_ref[m]
    gt = jax.lax.dot_general(g, eye_ref[...], (((0,), (0,)), ((), ())),
                             preferred_element_type=jnp.float32)
    out_ref[...] = (gt + p[:, None]).reshape(1, EMBED, BCH)


def kernel(x, token_table, pos_table):
    xt = jnp.transpose(x.astype(jnp.int32))

    mesh = plsc.VectorSubcoreMesh(core_axis_name="c", subcore_axis_name="s")
    gather = functools.partial(
        pl.kernel,
        mesh=mesh,
        compiler_params=pltpu.CompilerParams(use_tc_tiling_on_sc=False),
        out_type=jax.ShapeDtypeStruct((MAXLEN * BATCH, EMBED), jnp.float32),
        scratch_types=[pltpu.VMEM((BCH,), jnp.int32) for _ in range(NBUF)]
        + [pltpu.VMEM((BCH, EMBED), jnp.float32) for _ in range(NBUF)]
        + [pltpu.SemaphoreType.DMA for _ in range(3 * NBUF)],
    )(_gather_body)
    mid = gather(xt, token_table)
    mid3 = jnp.reshape(mid, (MAXLEN * BATCH // 2, 2, EMBED))

    out_t = pl.pallas_call(
        _addpos_body,
        grid=(ITEMS,),
        in_specs=[
            pl.BlockSpec((BCH // 2, 2, EMBED), lambda i: (i, 0, 0)),
            pl.BlockSpec((MAXLEN, EMBED), lambda i: (0, 0)),
            pl.BlockSpec((BCH, BCH), lambda i: (0, 0)),
        ],
        out_specs=pl.BlockSpec((1, EMBED, BCH),
                               lambda i: (i // NBC, 0, i % NBC)),
        out_shape=jax.ShapeDtypeStruct((MAXLEN, EMBED, BATCH), jnp.float32),
    )(mid3, pos_table, jnp.eye(BCH, dtype=jnp.float32))

    return jnp.transpose(out_t, (2, 0, 1))


# speedup vs baseline: 1.7189x; 1.7189x over previous
import functools

import jax
LANES = 16
import jax.numpy as jnp
from jax import lax
from jax.experimental import pallas as pl
from jax.experimental.pallas import tpu as pltpu
from jax.experimental.pallas import tpu_sc as plsc

MAXLEN = 200
EMBED = 64
BATCH = 4096
VOCAB = 100000

_info = plsc.get_sparse_core_info()
NC = _info.num_cores
NS = _info.num_subcores
NW = NC * NS

BCH = 512
NBC = BATCH // BCH
ITEMS = MAXLEN * NBC
IPW = ITEMS // NW
NBUF = 3
ROUNDS = IPW // NBUF
REM = IPW - ROUNDS * NBUF


def _gather_body(xt_hbm, tok_hbm, mid_hbm,
                 i0, i1, i2, r0, r1, r2,
                 g0, g1, g2, w0, w1, w2, s0, s1, s2):
    idx = (i0, i1, i2)
    rows = (r0, r1, r2)
    gsem = (g0, g1, g2)
    wsem = (w0, w1, w2)
    isem = (s0, s1, s2)

    wid = lax.axis_index("s") * NC + lax.axis_index("c")
    ibase = wid * IPW

    def fire_idx(p, it):
        m = it // NBC
        c = it % NBC
        pltpu.async_copy(xt_hbm.at[m, pl.ds(c * BCH, BCH)], idx[p], isem[p])

    def fire_gather(p, it):
        del it
        pltpu.make_async_copy(xt_hbm.at[0, pl.ds(0, BCH)], idx[p],
                              isem[p]).wait()
        pltpu.async_copy(tok_hbm.at[idx[p]], rows[p], gsem[p])

    def fire_wb(p, it):
        pltpu.make_async_copy(tok_hbm.at[pl.ds(0, BCH)], rows[p],
                              gsem[p]).wait()
        m = it // NBC
        c = it % NBC
        base = m * BATCH + c * BCH
        pltpu.async_copy(rows[p], mid_hbm.at[pl.ds(base, BCH)], wsem[p])

    def drain_wb(p):
        pltpu.make_async_copy(rows[p], mid_hbm.at[pl.ds(0, BCH)],
                              wsem[p]).wait()

    fire_idx(0, ibase)
    fire_idx(1, ibase + 1)
    fire_gather(0, ibase)

    def round_body(rnd, _):
        for p in range(NBUF):
            jj = rnd * NBUF + p
            it = ibase + jj
            q = (p + 1) % NBUF
            s = (p + 2) % NBUF

            def _prep(s=s, it=it, jj=jj):
                pl.when(jj >= 1)(lambda: drain_wb(s))
                fire_idx(s, it + 2)

            def _next(q=q, it=it):
                fire_gather(q, it + 1)

            pl.when(jj + 2 < IPW)(_prep)
            pl.when(jj + 1 < IPW)(_next)
            fire_wb(p, it)
        return ()

    lax.fori_loop(0, ROUNDS, round_body, ())
    for j in range(REM):
        jj = ROUNDS * NBUF + j
        p = jj % NBUF
        q = (p + 1) % NBUF
        if j + 1 < REM:
            pl.when(True)(lambda q=q: fire_gather(q, ibase + jj + 1))
        fire_wb(p, ibase + jj)

    for p in range(NBUF):
        drain_wb(p)


def _addpos_body(mid_ref, pos_ref, eye_ref, out_ref):
    m = pl.program_id(0) // NBC
    g = mid_ref[...]
    p = pos_ref[m]
    dn = (((0,), (0,)), ((), ()))
    gt = (jax.lax.dot_general(g[:, :EMBED], eye_ref[: BCH // 2], dn,
                              preferred_element_type=jnp.float32)
          + jax.lax.dot_general(g[:, EMBED:], eye_ref[BCH // 2:], dn,
                                preferred_element_type=jnp.float32))
    out_ref[...] = (gt + p[:, None]).reshape(1, EMBED, BCH)


def kernel(x, token_table, pos_table):
    xt = jnp.transpose(x.astype(jnp.int32))
    xt = (xt.reshape(MAXLEN, NBC, 2, BCH // 2)
            .transpose(0, 1, 3, 2)
            .reshape(MAXLEN, BATCH))

    mesh = plsc.VectorSubcoreMesh(core_axis_name="c", subcore_axis_name="s")
    gather = functools.partial(
        pl.kernel,
        mesh=mesh,
        compiler_params=pltpu.CompilerParams(use_tc_tiling_on_sc=False),
        out_type=jax.ShapeDtypeStruct((MAXLEN * BATCH, EMBED), jnp.float32),
        scratch_types=[pltpu.VMEM((BCH,), jnp.int32) for _ in range(NBUF)]
        + [pltpu.VMEM((BCH, EMBED), jnp.float32) for _ in range(NBUF)]
        + [pltpu.SemaphoreType.DMA for _ in range(3 * NBUF)],
    )(_gather_body)
    mid = gather(xt, token_table)
    mid128 = jnp.reshape(mid, (MAXLEN * BATCH // 2, 2 * EMBED))

    out_t = pl.pallas_call(
        _addpos_body,
        grid=(ITEMS,),
        in_specs=[
            pl.BlockSpec((BCH // 2, 2 * EMBED), lambda i: (i, 0)),
            pl.BlockSpec((MAXLEN, EMBED), lambda i: (0, 0)),
            pl.BlockSpec((BCH, BCH), lambda i: (0, 0)),
        ],
        out_specs=pl.BlockSpec((1, EMBED, BCH),
                               lambda i: (i // NBC, 0, i % NBC)),
        out_shape=jax.ShapeDtypeStruct((MAXLEN, EMBED, BATCH), jnp.float32),
    )(mid128, pos_table, jnp.eye(BCH, dtype=jnp.float32))

    return jnp.transpose(out_t, (2, 0, 1))


# speedup vs baseline: 1.8073x; 1.0514x over previous
import functools

import jax
LANES = 16
import jax.numpy as jnp
from jax import lax
from jax.experimental import pallas as pl
from jax.experimental.pallas import tpu as pltpu
from jax.experimental.pallas import tpu_sc as plsc

MAXLEN = 200
EMBED = 64
BATCH = 4096
VOCAB = 100000

_info = plsc.get_sparse_core_info()
NC = _info.num_cores
NS = _info.num_subcores
NW = NC * NS

BCH = 512
NBC = BATCH // BCH
ITEMS = MAXLEN * NBC
IPW = ITEMS // NW
NBUF = 3
ROUNDS = IPW // NBUF
REM = IPW - ROUNDS * NBUF


def _gather_body(xt_hbm, tok_hbm, mid_hbm,
                 i0, i1, i2, r0, r1, r2,
                 g0, g1, g2, w0, w1, w2, s0, s1, s2):
    idx = (i0, i1, i2)
    rows = (r0, r1, r2)
    gsem = (g0, g1, g2)
    wsem = (w0, w1, w2)
    isem = (s0, s1, s2)

    wid = lax.axis_index("s") * NC + lax.axis_index("c")
    ibase = wid * IPW

    def fire_idx(p, it):
        m = it // NBC
        c = it % NBC
        pltpu.async_copy(xt_hbm.at[m, pl.ds(c * BCH, BCH)], idx[p], isem[p])

    def fire_gather(p, it):
        del it
        pltpu.make_async_copy(xt_hbm.at[0, pl.ds(0, BCH)], idx[p],
                              isem[p]).wait()
        pltpu.async_copy(tok_hbm.at[idx[p]], rows[p], gsem[p])

    def fire_wb(p, it):
        pltpu.make_async_copy(tok_hbm.at[pl.ds(0, BCH)], rows[p],
                              gsem[p]).wait()
        m = it // NBC
        c = it % NBC
        base = m * BATCH + c * BCH
        pltpu.async_copy(rows[p], mid_hbm.at[pl.ds(base, BCH)], wsem[p])

    def drain_wb(p):
        pltpu.make_async_copy(rows[p], mid_hbm.at[pl.ds(0, BCH)],
                              wsem[p]).wait()

    fire_idx(0, ibase)
    fire_idx(1, ibase + 1)
    fire_gather(0, ibase)

    def round_body(rnd, _):
        for p in range(NBUF):
            jj = rnd * NBUF + p
            it = ibase + jj
            q = (p + 1) % NBUF
            s = (p + 2) % NBUF

            def _prep(s=s, it=it, jj=jj):
                pl.when(jj >= 1)(lambda: drain_wb(s))
                fire_idx(s, it + 2)

            def _next(q=q, it=it):
                fire_gather(q, it + 1)

            pl.when(jj + 2 < IPW)(_prep)
            pl.when(jj + 1 < IPW)(_next)
            fire_wb(p, it)
        return ()

    lax.fori_loop(0, ROUNDS, round_body, ())
    for j in range(REM):
        jj = ROUNDS * NBUF + j
        p = jj % NBUF
        q = (p + 1) % NBUF
        if j + 1 < REM:
            pl.when(True)(lambda q=q: fire_gather(q, ibase + jj + 1))
        fire_wb(p, ibase + jj)

    for p in range(NBUF):
        drain_wb(p)


def _addpos_body(mid_ref, pos_ref, eye_ref, out_ref):
    m = pl.program_id(0) // NBC
    g = mid_ref[...]
    p = pos_ref[m]
    dn = (((0,), (0,)), ((), ()))
    gt = jax.lax.dot_general(g, eye_ref[...], dn,
                             preferred_element_type=jnp.float32)
    ga = gt[:EMBED] + p[:, None]
    gb = gt[EMBED:] + p[:, None]
    out_ref[...] = jnp.concatenate([ga, gb], axis=1).reshape(1, EMBED, BCH)


def kernel(x, token_table, pos_table):
    xt = jnp.transpose(x.astype(jnp.int32))
    xt = (xt.reshape(MAXLEN, NBC, 2, BCH // 2)
            .transpose(0, 1, 3, 2)
            .reshape(MAXLEN, BATCH))

    mesh = plsc.VectorSubcoreMesh(core_axis_name="c", subcore_axis_name="s")
    gather = functools.partial(
        pl.kernel,
        mesh=mesh,
        compiler_params=pltpu.CompilerParams(use_tc_tiling_on_sc=False),
        out_type=jax.ShapeDtypeStruct((MAXLEN * BATCH, EMBED), jnp.float32),
        scratch_types=[pltpu.VMEM((BCH,), jnp.int32) for _ in range(NBUF)]
        + [pltpu.VMEM((BCH, EMBED), jnp.float32) for _ in range(NBUF)]
        + [pltpu.SemaphoreType.DMA for _ in range(3 * NBUF)],
    )(_gather_body)
    mid = gather(xt, token_table)
    mid128 = jnp.reshape(mid, (MAXLEN * BATCH // 2, 2 * EMBED))

    out_t = pl.pallas_call(
        _addpos_body,
        grid=(ITEMS,),
        in_specs=[
            pl.BlockSpec((BCH // 2, 2 * EMBED), lambda i: (i, 0)),
            pl.BlockSpec((MAXLEN, EMBED), lambda i: (0, 0)),
            pl.BlockSpec((BCH // 2, BCH // 2), lambda i: (0, 0)),
        ],
        out_specs=pl.BlockSpec((1, EMBED, BCH),
                               lambda i: (i // NBC, 0, i % NBC)),
        out_shape=jax.ShapeDtypeStruct((MAXLEN, EMBED, BATCH), jnp.float32),
    )(mid128, pos_table, jnp.eye(BCH // 2, dtype=jnp.float32))

    return jnp.transpose(out_t, (2, 0, 1))


# speedup vs baseline: 4.8320x; 2.6736x over previous
import functools

import jax
LANES = 16
import jax.numpy as jnp
from jax import lax
from jax.experimental import pallas as pl
from jax.experimental.pallas import tpu as pltpu
from jax.experimental.pallas import tpu_sc as plsc

MAXLEN = 200
EMBED = 64
BATCH = 4096
VOCAB = 100000

_info = plsc.get_sparse_core_info()
NC = _info.num_cores
NS = _info.num_subcores
NW = NC * NS

BCH = 512
NBC = BATCH // BCH
ITEMS = MAXLEN * NBC
IPW = ITEMS // NW
NBUF = 3
ROUNDS = IPW // NBUF
REM = IPW - ROUNDS * NBUF


def _gather_body(xt_hbm, tok_hbm, mid_hbm,
                 i0, i1, i2, r0, r1, r2,
                 g0, g1, g2, w0, w1, w2, s0, s1, s2):
    idx = (i0, i1, i2)
    rows = (r0, r1, r2)
    gsem = (g0, g1, g2)
    wsem = (w0, w1, w2)
    isem = (s0, s1, s2)

    wid = lax.axis_index("s") * NC + lax.axis_index("c")
    ibase = wid * IPW

    def fire_idx(p, it):
        m = it // NBC
        c = it % NBC
        pltpu.async_copy(xt_hbm.at[m, pl.ds(c * BCH, BCH)], idx[p], isem[p])

    def fire_gather(p, it):
        del it
        pltpu.make_async_copy(xt_hbm.at[0, pl.ds(0, BCH)], idx[p],
                              isem[p]).wait()
        pltpu.async_copy(tok_hbm.at[idx[p]], rows[p], gsem[p])

    def fire_wb(p, it):
        pltpu.make_async_copy(tok_hbm.at[pl.ds(0, BCH)], rows[p],
                              gsem[p]).wait()
        m = it // NBC
        c = it % NBC
        base = m * BATCH + c * BCH
        pltpu.async_copy(rows[p], mid_hbm.at[pl.ds(base, BCH)], wsem[p])

    def drain_wb(p):
        pltpu.make_async_copy(rows[p], mid_hbm.at[pl.ds(0, BCH)],
                              wsem[p]).wait()

    fire_idx(0, ibase)
    fire_idx(1, ibase + 1)
    fire_gather(0, ibase)

    def round_body(rnd, _):
        for p in range(NBUF):
            jj = rnd * NBUF + p
            it = ibase + jj
            q = (p + 1) % NBUF
            s = (p + 2) % NBUF

            def _prep(s=s, it=it, jj=jj):
                pl.when(jj >= 1)(lambda: drain_wb(s))
                fire_idx(s, it + 2)

            def _next(q=q, it=it):
                fire_gather(q, it + 1)

            pl.when(jj + 2 < IPW)(_prep)
            pl.when(jj + 1 < IPW)(_next)
            fire_wb(p, it)
        return ()

    lax.fori_loop(0, ROUNDS, round_body, ())
    for j in range(REM):
        jj = ROUNDS * NBUF + j
        p = jj % NBUF
        q = (p + 1) % NBUF
        if j + 1 < REM:
            pl.when(True)(lambda q=q: fire_gather(q, ibase + jj + 1))
        fire_wb(p, ibase + jj)

    for p in range(NBUF):
        drain_wb(p)


def _addpos_body(mid_ref, pos_ref, eye_ref, out_ref):
    m = pl.program_id(0)
    p = pos_ref[m]
    eye = eye_ref[...]
    dn = (((0,), (0,)), ((), ()))
    for c in range(NBC):
        h = BCH // 2
        g = mid_ref[pl.ds(c * h, h), :]
        gt = jax.lax.dot_general(g, eye, dn,
                                 preferred_element_type=jnp.float32)
        ga = gt[:EMBED] + p[:, None]
        gb = gt[EMBED:] + p[:, None]
        out_ref[0, :, pl.ds(c * BCH, BCH)] = jnp.concatenate([ga, gb],
                                                             axis=1)


def kernel(x, token_table, pos_table):
    xt = jnp.transpose(x.astype(jnp.int32))
    xt = (xt.reshape(MAXLEN, NBC, 2, BCH // 2)
            .transpose(0, 1, 3, 2)
            .reshape(MAXLEN, BATCH))

    mesh = plsc.VectorSubcoreMesh(core_axis_name="c", subcore_axis_name="s")
    gather = functools.partial(
        pl.kernel,
        mesh=mesh,
        compiler_params=pltpu.CompilerParams(use_tc_tiling_on_sc=False),
        out_type=jax.ShapeDtypeStruct((MAXLEN * BATCH, EMBED), jnp.float32),
        scratch_types=[pltpu.VMEM((BCH,), jnp.int32) for _ in range(NBUF)]
        + [pltpu.VMEM((BCH, EMBED), jnp.float32) for _ in range(NBUF)]
        + [pltpu.SemaphoreType.DMA for _ in range(3 * NBUF)],
    )(_gather_body)
    mid = gather(xt, token_table)
    mid128 = jnp.reshape(mid, (MAXLEN * BATCH // 2, 2 * EMBED))

    out_t = pl.pallas_call(
        _addpos_body,
        grid=(MAXLEN,),
        in_specs=[
            pl.BlockSpec((BATCH // 2, 2 * EMBED), lambda m: (m, 0)),
            pl.BlockSpec((MAXLEN, EMBED), lambda m: (0, 0)),
            pl.BlockSpec((BCH // 2, BCH // 2), lambda m: (0, 0)),
        ],
        out_specs=pl.BlockSpec((1, EMBED, BATCH), lambda m: (m, 0, 0)),
        out_shape=jax.ShapeDtypeStruct((MAXLEN, EMBED, BATCH), jnp.float32),
    )(mid128, pos_table, jnp.eye(BCH // 2, dtype=jnp.float32))

    return jnp.transpose(out_t, (2, 0, 1))
